# Initial kernel scaffold; baseline (speedup 1.0000x reference)
import jax
import jax.numpy as jnp
from jax.experimental import pallas as pl


def kernel(y, neighbors_index, neighbors_row_splits, W, b):
    raise NotImplementedError("write your pallas kernel here")


# speedup vs baseline: 2.9894x; 2.9894x over previous
import functools

import jax
import jax.numpy as jnp
from jax import lax
from jax.experimental import pallas as pl
from jax.experimental.pallas import tpu as pltpu
from jax.experimental.pallas import tpu_sc as plsc

N_NODES = 10000
K_NBRS = 32
D = 128
L = 16
NW = 32
NPT = 320
NPAD = NW * NPT
CHUNK = 4
ROWS = CHUNK * K_NBRS
NCHUNK = NPT // CHUNK


def _sc_gather_sum(y, nidx_flat):
    mesh = plsc.VectorSubcoreMesh(core_axis_name="c", subcore_axis_name="s")

    @functools.partial(
        pl.kernel,
        out_type=jax.ShapeDtypeStruct((NPAD, D), jnp.float32),
        mesh=mesh,
        scratch_types=[
            pltpu.VMEM((ROWS,), jnp.int32),
            pltpu.VMEM((ROWS, D), jnp.float32),
            pltpu.VMEM((NPT, D), jnp.float32),
            pltpu.SemaphoreType.DMA,
        ],
    )
    def body(y_hbm, nidx_hbm, out_hbm, idx_v, gbuf, obuf, sem):
        wid = lax.axis_index("c") * 16 + lax.axis_index("s")
        node_base = wid * NPT

        def chunk_body(g, carry):
            idx_base = (node_base + g * CHUNK) * K_NBRS
            pltpu.sync_copy(nidx_hbm.at[pl.ds(idx_base, ROWS)], idx_v)
            pltpu.async_copy(y_hbm.at[idx_v], gbuf, sem).wait()
            for n in range(CHUNK):
                def red(r, accs):
                    row = n * K_NBRS + r
                    return tuple(
                        accs[j] + gbuf[row, pl.ds(j * L, L)] for j in range(D // L)
                    )
                accs = lax.fori_loop(
                    0, K_NBRS, red,
                    tuple(jnp.zeros((L,), jnp.float32) for _ in range(D // L)),
                )
                for j in range(D // L):
                    obuf[g * CHUNK + n, pl.ds(j * L, L)] = accs[j]
            return carry

        lax.fori_loop(0, NCHUNK, chunk_body, 0)
        pltpu.sync_copy(obuf, out_hbm.at[pl.ds(node_base, NPT)])

    return body(y, nidx_flat)


def _tc_combine(g, y, w_top, w_bot, b2d):
    blk = 400
    grid = (N_NODES // blk,)

    def body(g_ref, y_ref, wt_ref, wb_ref, b_ref, o_ref):
        acc = jnp.dot(g_ref[...], wt_ref[...], preferred_element_type=jnp.float32)
        acc += float(K_NBRS) * jnp.dot(
            y_ref[...], wb_ref[...], preferred_element_type=jnp.float32
        )
        o_ref[...] = acc + float(K_NBRS) * b_ref[...]

    return pl.pallas_call(
        body,
        grid=grid,
        in_specs=[
            pl.BlockSpec((blk, D), lambda i: (i, 0)),
            pl.BlockSpec((blk, D), lambda i: (i, 0)),
            pl.BlockSpec((D, D), lambda i: (0, 0)),
            pl.BlockSpec((D, D), lambda i: (0, 0)),
            pl.BlockSpec((1, D), lambda i: (0, 0)),
        ],
        out_specs=pl.BlockSpec((blk, D), lambda i: (i, 0)),
        out_shape=jax.ShapeDtypeStruct((N_NODES, D), jnp.float32),
    )(g, y, w_top, w_bot, b2d)


def kernel(y, neighbors_index, neighbors_row_splits, W, b):
    nidx = jnp.pad(neighbors_index, ((0, NPAD - N_NODES), (0, 0)))
    nidx_flat = nidx.reshape(-1)
    g = _sc_gather_sum(y, nidx_flat)[:N_NODES]
    return _tc_combine(g, y, W[:D], W[D:], b.reshape(1, D))


# speedup vs baseline: 3.4664x; 1.1595x over previous
import functools

import jax
import jax.numpy as jnp
from jax import lax
from jax.experimental import pallas as pl
from jax.experimental.pallas import tpu as pltpu
from jax.experimental.pallas import tpu_sc as plsc

N_NODES = 10000
K_NBRS = 32
D = 128
L = 16
NW = 32
NPT = 320
NPAD = NW * NPT
CHUNK = 4
ROWS = CHUNK * K_NBRS
NCHUNK = NPT // CHUNK


def _sc_gather_sum(y, nidx_flat):
    mesh = plsc.VectorSubcoreMesh(core_axis_name="c", subcore_axis_name="s")

    @functools.partial(
        pl.kernel,
        out_type=jax.ShapeDtypeStruct((NPAD, D), jnp.float32),
        mesh=mesh,
        scratch_types=[
            pltpu.VMEM((NPT * K_NBRS,), jnp.int32),
            pltpu.VMEM((2, ROWS, D), jnp.float32),
            pltpu.VMEM((NPT, D), jnp.float32),
            pltpu.SemaphoreType.DMA((2,)),
        ],
    )
    def body(y_hbm, nidx_hbm, out_hbm, idx_v, gbuf, obuf, sem):
        wid = lax.axis_index("c") * 16 + lax.axis_index("s")
        node_base = wid * NPT

        pltpu.sync_copy(nidx_hbm.at[pl.ds(node_base * K_NBRS, NPT * K_NBRS)], idx_v)

        def gather(g, p):
            return pltpu.make_async_copy(
                y_hbm.at[idx_v.at[pl.ds(g * ROWS, ROWS)]], gbuf.at[p], sem.at[p]
            )

        gather(0, 0).start()
        gather(1, 1).start()

        def chunk_body(g, carry):
            p = lax.rem(g, 2)
            gather(g, p).wait()
            for n in range(CHUNK):
                def red(r, accs):
                    row = n * K_NBRS + r
                    return tuple(
                        accs[j] + gbuf[p, row, pl.ds(j * L, L)]
                        for j in range(D // L)
                    )
                accs = lax.fori_loop(
                    0, K_NBRS, red,
                    tuple(jnp.zeros((L,), jnp.float32) for _ in range(D // L)),
                )
                for j in range(D // L):
                    obuf[g * CHUNK + n, pl.ds(j * L, L)] = accs[j]

            @pl.when(g + 2 < NCHUNK)
            def _():
                gather(g + 2, p).start()

            return carry

        lax.fori_loop(0, NCHUNK, chunk_body, 0)
        pltpu.sync_copy(obuf, out_hbm.at[pl.ds(node_base, NPT)])

    return body(y, nidx_flat)


def _tc_combine(g, y, w_top, w_bot, b2d):
    blk = 400
    grid = (N_NODES // blk,)

    def body(g_ref, y_ref, wt_ref, wb_ref, b_ref, o_ref):
        acc = jnp.dot(g_ref[...], wt_ref[...], preferred_element_type=jnp.float32)
        acc += float(K_NBRS) * jnp.dot(
            y_ref[...], wb_ref[...], preferred_element_type=jnp.float32
        )
        o_ref[...] = acc + float(K_NBRS) * b_ref[...]

    return pl.pallas_call(
        body,
        grid=grid,
        in_specs=[
            pl.BlockSpec((blk, D), lambda i: (i, 0)),
            pl.BlockSpec((blk, D), lambda i: (i, 0)),
            pl.BlockSpec((D, D), lambda i: (0, 0)),
            pl.BlockSpec((D, D), lambda i: (0, 0)),
            pl.BlockSpec((1, D), lambda i: (0, 0)),
        ],
        out_specs=pl.BlockSpec((blk, D), lambda i: (i, 0)),
        out_shape=jax.ShapeDtypeStruct((N_NODES, D), jnp.float32),
    )(g, y, w_top, w_bot, b2d)


def kernel(y, neighbors_index, neighbors_row_splits, W, b):
    nidx = jnp.pad(neighbors_index, ((0, NPAD - N_NODES), (0, 0)))
    nidx_flat = nidx.reshape(-1)
    g = _sc_gather_sum(y, nidx_flat)[:N_NODES]
    return _tc_combine(g, y, W[:D], W[D:], b.reshape(1, D))


# speedup vs baseline: 7.6676x; 2.2120x over previous
import functools

import jax
import jax.numpy as jnp
from jax import lax
from jax.experimental import pallas as pl
from jax.experimental.pallas import tpu as pltpu
from jax.experimental.pallas import tpu_sc as plsc

N_NODES = 10000
K_NBRS = 32
D = 128
L = 16
DW = D // 2
NW = 32
NPT = 320
NPAD = NW * NPT
CHUNK = 4
ROWS = CHUNK * K_NBRS
NCHUNK = NPT // CHUNK
NBUF = 4


def _sc_gather_sum(y_pack, nidx_flat):
    mesh = plsc.VectorSubcoreMesh(core_axis_name="c", subcore_axis_name="s")

    @functools.partial(
        pl.kernel,
        out_type=jax.ShapeDtypeStruct((NPAD, D), jnp.float32),
        mesh=mesh,
        compiler_params=pltpu.CompilerParams(use_tc_tiling_on_sc=False),
        scratch_types=[
            pltpu.VMEM((NPT * K_NBRS,), jnp.int32),
            pltpu.VMEM((NBUF, ROWS, DW), jnp.int32),
            pltpu.VMEM((NPT, D), jnp.float32),
            pltpu.SemaphoreType.DMA((NBUF,)),
        ],
    )
    def body(y_hbm, nidx_hbm, out_hbm, idx_v, gbuf, obuf, sem):
        sid = lax.axis_index("s")
        wid = lax.axis_index("c") * 16 + sid
        node_base = wid * NPT

        pltpu.sync_copy(nidx_hbm.at[pl.ds(node_base * K_NBRS, NPT * K_NBRS)], idx_v)

        def gather(g, p):
            return pltpu.make_async_copy(
                y_hbm.at[idx_v.at[pl.ds(g * ROWS, ROWS)]], gbuf.at[p], sem.at[p]
            )

        for i in range(NBUF):
            gather(i, i).start()

        def chunk_body(g, carry):
            p = lax.rem(g, NBUF)
            gather(g, p).wait()
            for n in range(CHUNK):
                def red(r, accs):
                    row = n * K_NBRS + r
                    new = list(accs)
                    for grp in range(DW // L):
                        w = gbuf[p, row, pl.ds(grp * L, L)]
                        a = lax.bitcast_convert_type(
                            lax.shift_left(w, jnp.full((L,), 16, jnp.int32)),
                            jnp.float32,
                        )
                        b = lax.bitcast_convert_type(
                            lax.bitwise_and(w, jnp.full((L,), -65536, jnp.int32)),
                            jnp.float32,
                        )
                        new[2 * grp] = new[2 * grp] + a
                        new[2 * grp + 1] = new[2 * grp + 1] + b
                    return tuple(new)
                accs = lax.fori_loop(
                    0, K_NBRS, red,
                    tuple(jnp.zeros((L,), jnp.float32) for _ in range(D // L)),
                )
                for j in range(D // L):
                    obuf[g * CHUNK + n, pl.ds(j * L, L)] = accs[j]

            @pl.when(g + NBUF < NCHUNK)
            def _():
                gather(g + NBUF, p).start()

            return carry

        lax.fori_loop(0, NCHUNK, chunk_body, 0)
        pltpu.sync_copy(obuf, out_hbm.at[pl.ds(node_base, NPT)])

    return body(y_pack, nidx_flat)


def _tc_combine(g, y, w_top, w_bot, b2d):
    blk = 400
    grid = (N_NODES // blk,)

    def body(g_ref, y_ref, wt_ref, wb_ref, b_ref, o_ref):
        acc = jnp.dot(g_ref[...], wt_ref[...], preferred_element_type=jnp.float32)
        acc += float(K_NBRS) * jnp.dot(
            y_ref[...], wb_ref[...], preferred_element_type=jnp.float32
        )
        o_ref[...] = acc + float(K_NBRS) * b_ref[...]

    return pl.pallas_call(
        body,
        grid=grid,
        in_specs=[
            pl.BlockSpec((blk, D), lambda i: (i, 0)),
            pl.BlockSpec((blk, D), lambda i: (i, 0)),
            pl.BlockSpec((D, D), lambda i: (0, 0)),
            pl.BlockSpec((D, D), lambda i: (0, 0)),
            pl.BlockSpec((1, D), lambda i: (0, 0)),
        ],
        out_specs=pl.BlockSpec((blk, D), lambda i: (i, 0)),
        out_shape=jax.ShapeDtypeStruct((N_NODES, D), jnp.float32),
    )(g, y, w_top, w_bot, b2d)


def kernel(y, neighbors_index, neighbors_row_splits, W, b):
    nidx = jnp.pad(neighbors_index, ((0, NPAD - N_NODES), (0, 0)))
    nidx_flat = nidx.reshape(-1)
    y_pad = jnp.pad(y, ((0, NPAD - N_NODES), (0, 0))).astype(jnp.bfloat16)
    y_grp = y_pad.reshape(NPAD, D // 32, 2, L)
    y_pair = jnp.stack([y_grp[:, :, 0, :], y_grp[:, :, 1, :]], axis=-1)
    y_pack = jax.lax.bitcast_convert_type(y_pair, jnp.int32).reshape(NPAD, DW)
    g = _sc_gather_sum(y_pack, nidx_flat)[:N_NODES]
    return _tc_combine(g, y, W[:D], W[D:], b.reshape(1, D))


# speedup vs baseline: 7.7931x; 1.0164x over previous
import functools

import jax
import jax.numpy as jnp
from jax import lax
from jax.experimental import pallas as pl
from jax.experimental.pallas import tpu as pltpu
from jax.experimental.pallas import tpu_sc as plsc

N_NODES = 10000
K_NBRS = 32
D = 128
L = 16
DW = D // 2
NW = 32
NPT = 320
NPAD = NW * NPT
CHUNK = 4
ROWS = CHUNK * K_NBRS
NCHUNK = NPT // CHUNK
NBUF = 8


def _sc_gather_sum(y_pack, nidx_flat):
    mesh = plsc.VectorSubcoreMesh(core_axis_name="c", subcore_axis_name="s")

    @functools.partial(
        pl.kernel,
        out_type=jax.ShapeDtypeStruct((NPAD, D), jnp.float32),
        mesh=mesh,
        compiler_params=pltpu.CompilerParams(use_tc_tiling_on_sc=False),
        scratch_types=[
            pltpu.VMEM((NPT * K_NBRS,), jnp.int32),
            pltpu.VMEM((NBUF, ROWS, DW), jnp.int32),
            pltpu.VMEM((NPT, D), jnp.float32),
            pltpu.SemaphoreType.DMA((NBUF,)),
        ],
    )
    def body(y_hbm, nidx_hbm, out_hbm, idx_v, gbuf, obuf, sem):
        sid = lax.axis_index("s")
        wid = lax.axis_index("c") * 16 + sid
        node_base = wid * NPT

        pltpu.sync_copy(nidx_hbm.at[pl.ds(node_base * K_NBRS, NPT * K_NBRS)], idx_v)

        def gather(g, p):
            return pltpu.make_async_copy(
                y_hbm.at[idx_v.at[pl.ds(g * ROWS, ROWS)]], gbuf.at[p], sem.at[p]
            )

        for i in range(NBUF):
            gather(i, i).start()

        def chunk_body(g, carry):
            p = lax.rem(g, NBUF)
            gather(g, p).wait()
            for n in range(CHUNK):
                def red(r, accs):
                    row = n * K_NBRS + r
                    new = list(accs)
                    for grp in range(DW // L):
                        w = gbuf[p, row, pl.ds(grp * L, L)]
                        a = lax.bitcast_convert_type(
                            lax.shift_left(w, jnp.full((L,), 16, jnp.int32)),
                            jnp.float32,
                        )
                        b = lax.bitcast_convert_type(
                            lax.bitwise_and(w, jnp.full((L,), -65536, jnp.int32)),
                            jnp.float32,
                        )
                        new[2 * grp] = new[2 * grp] + a
                        new[2 * grp + 1] = new[2 * grp + 1] + b
                    return tuple(new)
                accs = lax.fori_loop(
                    0, K_NBRS, red,
                    tuple(jnp.zeros((L,), jnp.float32) for _ in range(D // L)),
                )
                for j in range(D // L):
                    obuf[g * CHUNK + n, pl.ds(j * L, L)] = accs[j]

            @pl.when(g + NBUF < NCHUNK)
            def _():
                gather(g + NBUF, p).start()

            return carry

        lax.fori_loop(0, NCHUNK, chunk_body, 0)
        pltpu.sync_copy(obuf, out_hbm.at[pl.ds(node_base, NPT)])

    return body(y_pack, nidx_flat)


def _tc_combine(g, y, w_top, w_bot, b2d):
    blk = 400
    grid = (N_NODES // blk,)

    def body(g_ref, y_ref, wt_ref, wb_ref, b_ref, o_ref):
        acc = jnp.dot(g_ref[...], wt_ref[...], preferred_element_type=jnp.float32)
        acc += float(K_NBRS) * jnp.dot(
            y_ref[...], wb_ref[...], preferred_element_type=jnp.float32
        )
        o_ref[...] = acc + float(K_NBRS) * b_ref[...]

    return pl.pallas_call(
        body,
        grid=grid,
        in_specs=[
            pl.BlockSpec((blk, D), lambda i: (i, 0)),
            pl.BlockSpec((blk, D), lambda i: (i, 0)),
            pl.BlockSpec((D, D), lambda i: (0, 0)),
            pl.BlockSpec((D, D), lambda i: (0, 0)),
            pl.BlockSpec((1, D), lambda i: (0, 0)),
        ],
        out_specs=pl.BlockSpec((blk, D), lambda i: (i, 0)),
        out_shape=jax.ShapeDtypeStruct((N_NODES, D), jnp.float32),
    )(g, y, w_top, w_bot, b2d)


def kernel(y, neighbors_index, neighbors_row_splits, W, b):
    nidx = jnp.pad(neighbors_index, ((0, NPAD - N_NODES), (0, 0)))
    nidx_flat = nidx.reshape(-1)
    y_pad = jnp.pad(y, ((0, NPAD - N_NODES), (0, 0))).astype(jnp.bfloat16)
    y_grp = y_pad.reshape(NPAD, D // 32, 2, L)
    y_pair = jnp.stack([y_grp[:, :, 0, :], y_grp[:, :, 1, :]], axis=-1)
    y_pack = jax.lax.bitcast_convert_type(y_pair, jnp.int32).reshape(NPAD, DW)
    g = _sc_gather_sum(y_pack, nidx_flat)[:N_NODES]
    return _tc_combine(g, y, W[:D], W[D:], b.reshape(1, D))


# speedup vs baseline: 20.6436x; 2.6490x over previous
import functools

import jax
import jax.numpy as jnp
from jax import lax
from jax.experimental import pallas as pl
from jax.experimental.pallas import tpu as pltpu
from jax.experimental.pallas import tpu_sc as plsc

N_NODES = 10000
K_NBRS = 32
D = 128
L = 16
DW = D // 2
NW = 32
NPT = 320
CHUNK = 4
ROWS = CHUNK * K_NBRS
NCHUNK = NPT // CHUNK
NBUF = 2


def _sc_gather_sum(y_pack, nidx_flat):
    mesh = plsc.VectorSubcoreMesh(core_axis_name="c", subcore_axis_name="s")

    @functools.partial(
        pl.kernel,
        out_type=jax.ShapeDtypeStruct((N_NODES, D), jnp.float32),
        mesh=mesh,
        compiler_params=pltpu.CompilerParams(use_tc_tiling_on_sc=False),
        scratch_types=[
            pltpu.VMEM((NPT * K_NBRS,), jnp.int32),
            pltpu.VMEM((NBUF, ROWS, DW), jnp.int32),
            pltpu.VMEM((NPT, D), jnp.float32),
            pltpu.VMEM_SHARED((N_NODES, DW), jnp.int32),
            pltpu.SemaphoreType.DMA((NBUF,)),
        ],
    )
    def body(y_hbm, nidx_hbm, out_hbm, idx_v, gbuf, obuf, yspm, sem):
        sid = lax.axis_index("s")
        wid = lax.axis_index("c") * 16 + sid
        node_base = jnp.minimum(wid * NPT, N_NODES - NPT)

        rows_a = 624
        @pl.when(sid < 15)
        def _():
            pltpu.sync_copy(
                y_hbm.at[pl.ds(sid * rows_a, rows_a)],
                yspm.at[pl.ds(sid * rows_a, rows_a)],
            )
        @pl.when(sid == 15)
        def _():
            pltpu.sync_copy(
                y_hbm.at[pl.ds(15 * rows_a, N_NODES - 15 * rows_a)],
                yspm.at[pl.ds(15 * rows_a, N_NODES - 15 * rows_a)],
            )
        pltpu.sync_copy(
            nidx_hbm.at[pl.ds(node_base * K_NBRS, NPT * K_NBRS)], idx_v
        )
        plsc.subcore_barrier()

        def gather(g, p):
            return pltpu.make_async_copy(
                yspm.at[idx_v.at[pl.ds(g * ROWS, ROWS)]], gbuf.at[p], sem.at[p]
            )

        for i in range(NBUF):
            gather(i, i).start()

        def chunk_body(g, carry):
            p = lax.rem(g, NBUF)
            gather(g, p).wait()
            for n in range(CHUNK):
                def red(r, accs):
                    row = n * K_NBRS + r
                    new = list(accs)
                    for grp in range(DW // L):
                        w = gbuf[p, row, pl.ds(grp * L, L)]
                        a = lax.bitcast_convert_type(
                            lax.shift_left(w, jnp.full((L,), 16, jnp.int32)),
                            jnp.float32,
                        )
                        b = lax.bitcast_convert_type(
                            lax.bitwise_and(w, jnp.full((L,), -65536, jnp.int32)),
                            jnp.float32,
                        )
                        new[2 * grp] = new[2 * grp] + a
                        new[2 * grp + 1] = new[2 * grp + 1] + b
                    return tuple(new)
                accs = lax.fori_loop(
                    0, K_NBRS, red,
                    tuple(jnp.zeros((L,), jnp.float32) for _ in range(D // L)),
                )
                for j in range(D // L):
                    obuf[g * CHUNK + n, pl.ds(j * L, L)] = accs[j]

            @pl.when(g + NBUF < NCHUNK)
            def _():
                gather(g + NBUF, p).start()

            return carry

        lax.fori_loop(0, NCHUNK, chunk_body, 0)
        pltpu.sync_copy(obuf, out_hbm.at[pl.ds(node_base, NPT)])

    return body(y_pack, nidx_flat)


def _tc_combine(g, y, w_top, w_bot, b2d):
    blk = 1000
    grid = (N_NODES // blk,)

    def body(g_ref, y_ref, wt_ref, wb_ref, b_ref, o_ref):
        acc = jnp.dot(g_ref[...], wt_ref[...], preferred_element_type=jnp.float32)
        acc += float(K_NBRS) * jnp.dot(
            y_ref[...], wb_ref[...], preferred_element_type=jnp.float32
        )
        o_ref[...] = acc + float(K_NBRS) * b_ref[...]

    return pl.pallas_call(
        body,
        grid=grid,
        in_specs=[
            pl.BlockSpec((blk, D), lambda i: (i, 0)),
            pl.BlockSpec((blk, D), lambda i: (i, 0)),
            pl.BlockSpec((D, D), lambda i: (0, 0)),
            pl.BlockSpec((D, D), lambda i: (0, 0)),
            pl.BlockSpec((1, D), lambda i: (0, 0)),
        ],
        out_specs=pl.BlockSpec((blk, D), lambda i: (i, 0)),
        out_shape=jax.ShapeDtypeStruct((N_NODES, D), jnp.float32),
    )(g, y, w_top, w_bot, b2d)


def kernel(y, neighbors_index, neighbors_row_splits, W, b):
    nidx_flat = neighbors_index.reshape(-1)
    y_bf = y.astype(jnp.bfloat16)
    y_grp = y_bf.reshape(N_NODES, D // 32, 2, L)
    y_pair = jnp.stack([y_grp[:, :, 0, :], y_grp[:, :, 1, :]], axis=-1)
    y_pack = jax.lax.bitcast_convert_type(y_pair, jnp.int32).reshape(N_NODES, DW)
    g = _sc_gather_sum(y_pack, nidx_flat)
    return _tc_combine(g, y, W[:D], W[D:], b.reshape(1, D))


# speedup vs baseline: 21.2800x; 1.0308x over previous
import functools

import jax
import jax.numpy as jnp
from jax import lax
from jax.experimental import pallas as pl
from jax.experimental.pallas import tpu as pltpu
from jax.experimental.pallas import tpu_sc as plsc

N_NODES = 10000
K_NBRS = 32
D = 128
L = 16
DW = D // 2
NW = 32
NPT = 320
CHUNK = 4
ROWS = CHUNK * K_NBRS
NCHUNK = NPT // CHUNK
NBUF = 2


def _sc_gather_sum(y_pack, nidx_flat):
    mesh = plsc.VectorSubcoreMesh(core_axis_name="c", subcore_axis_name="s")

    @functools.partial(
        pl.kernel,
        out_type=jax.ShapeDtypeStruct((N_NODES, D), jnp.float32),
        mesh=mesh,
        compiler_params=pltpu.CompilerParams(use_tc_tiling_on_sc=False),
        scratch_types=[
            pltpu.VMEM((NPT * K_NBRS,), jnp.int32),
            pltpu.VMEM((NBUF, ROWS, DW), jnp.int32),
            pltpu.VMEM((NPT, D), jnp.float32),
            pltpu.VMEM_SHARED((N_NODES, DW), jnp.int32),
            pltpu.SemaphoreType.DMA((NBUF,)),
        ],
    )
    def body(y_hbm, nidx_hbm, out_hbm, idx_v, gbuf, obuf, yspm, sem):
        sid = lax.axis_index("s")
        wid = lax.axis_index("c") * 16 + sid
        node_base = jnp.minimum(wid * NPT, N_NODES - NPT)

        rows_a = 624
        @pl.when(sid < 15)
        def _():
            pltpu.sync_copy(
                y_hbm.at[pl.ds(sid * rows_a, rows_a)],
                yspm.at[pl.ds(sid * rows_a, rows_a)],
            )
        @pl.when(sid == 15)
        def _():
            pltpu.sync_copy(
                y_hbm.at[pl.ds(15 * rows_a, N_NODES - 15 * rows_a)],
                yspm.at[pl.ds(15 * rows_a, N_NODES - 15 * rows_a)],
            )
        pltpu.sync_copy(
            nidx_hbm.at[pl.ds(node_base * K_NBRS, NPT * K_NBRS)], idx_v
        )
        plsc.subcore_barrier()

        def gather(g, p):
            return pltpu.make_async_copy(
                yspm.at[idx_v.at[pl.ds(g * ROWS, ROWS)]], gbuf.at[p], sem.at[p]
            )

        for i in range(NBUF):
            gather(i, i).start()

        def chunk_body(g, carry):
            p = lax.rem(g, NBUF)
            gather(g, p).wait()
            for n in range(CHUNK):
                def red(r, accs):
                    row = n * K_NBRS + r
                    new = list(accs)
                    for grp in range(DW // L):
                        w = gbuf[p, row, pl.ds(grp * L, L)]
                        a = lax.bitcast_convert_type(
                            lax.shift_left(w, jnp.full((L,), 16, jnp.int32)),
                            jnp.float32,
                        )
                        b = lax.bitcast_convert_type(
                            lax.bitwise_and(w, jnp.full((L,), -65536, jnp.int32)),
                            jnp.float32,
                        )
                        new[2 * grp] = new[2 * grp] + a
                        new[2 * grp + 1] = new[2 * grp + 1] + b
                    return tuple(new)
                accs = lax.fori_loop(
                    0, K_NBRS, red,
                    tuple(jnp.zeros((L,), jnp.float32) for _ in range(D // L)),
                )
                for j in range(D // L):
                    obuf[g * CHUNK + n, pl.ds(j * L, L)] = accs[j]

            @pl.when(g + NBUF < NCHUNK)
            def _():
                gather(g + NBUF, p).start()

            return carry

        lax.fori_loop(0, NCHUNK, chunk_body, 0)
        pltpu.sync_copy(obuf, out_hbm.at[pl.ds(node_base, NPT)])

    return body(y_pack, nidx_flat)


def _tc_self(y, w_bot, b2d):
    blk = 2000
    grid = (N_NODES // blk,)

    def body(y_ref, wb_ref, b_ref, o_ref):
        o_ref[...] = float(K_NBRS) * jnp.dot(
            y_ref[...], wb_ref[...], preferred_element_type=jnp.float32
        ) + float(K_NBRS) * b_ref[...]

    return pl.pallas_call(
        body,
        grid=grid,
        in_specs=[
            pl.BlockSpec((blk, D), lambda i: (i, 0)),
            pl.BlockSpec((D, D), lambda i: (0, 0)),
            pl.BlockSpec((1, D), lambda i: (0, 0)),
        ],
        out_specs=pl.BlockSpec((blk, D), lambda i: (i, 0)),
        out_shape=jax.ShapeDtypeStruct((N_NODES, D), jnp.float32),
    )(y, w_bot, b2d)


def _tc_final(g, s, w_top):
    blk = 2000
    grid = (N_NODES // blk,)

    def body(g_ref, s_ref, wt_ref, o_ref):
        o_ref[...] = jnp.dot(
            g_ref[...], wt_ref[...], preferred_element_type=jnp.float32
        ) + s_ref[...]

    return pl.pallas_call(
        body,
        grid=grid,
        in_specs=[
            pl.BlockSpec((blk, D), lambda i: (i, 0)),
            pl.BlockSpec((blk, D), lambda i: (i, 0)),
            pl.BlockSpec((D, D), lambda i: (0, 0)),
        ],
        out_specs=pl.BlockSpec((blk, D), lambda i: (i, 0)),
        out_shape=jax.ShapeDtypeStruct((N_NODES, D), jnp.float32),
    )(g, s, w_top)


def kernel(y, neighbors_index, neighbors_row_splits, W, b):
    nidx_flat = neighbors_index.reshape(-1)
    y_bf = y.astype(jnp.bfloat16)
    y_grp = y_bf.reshape(N_NODES, D // 32, 2, L)
    y_pair = jnp.stack([y_grp[:, :, 0, :], y_grp[:, :, 1, :]], axis=-1)
    y_pack = jax.lax.bitcast_convert_type(y_pair, jnp.int32).reshape(N_NODES, DW)
    g = _sc_gather_sum(y_pack, nidx_flat)
    s = _tc_self(y, W[D:], b.reshape(1, D))
    return _tc_final(g, s, W[:D])
